# speedup vs baseline: 5.4245x; 5.4245x over previous
import functools

import jax
import jax.numpy as jnp
import numpy as np
from jax.experimental import pallas as pl
from jax.experimental.pallas import tpu as pltpu

_B, _N, _C = 8, 1568, 384
_CN, _K = 196, 9
_T = 224


def _ctm_body(x_ref, wt_ref, b_ref, noise_ref, out_ref, dm_ref, *,
              n, c, cn, k, t, prec_g):
    f32 = jnp.float32
    sqrt_c = np.float32(c ** 0.5)
    xb = x_ref[0]
    sq = jnp.sum(xb * xb, axis=1)

    nt = n // t
    lane_t = jax.lax.broadcasted_iota(jnp.int32, (t, n), 1)

    dens_parts = []
    rmax = jnp.full((), -jnp.inf, f32)
    for ti in range(nt):
        rows = xb[ti * t:(ti + 1) * t]
        g = jax.lax.dot_general(rows, xb, (((1,), (1,)), ((), ())),
                                precision=prec_g, preferred_element_type=f32)
        d2 = sq[ti * t:(ti + 1) * t][:, None] + sq[None, :] - 2.0 * g
        dm = jnp.sqrt(jnp.maximum(d2, 0.0)) / sqrt_c
        dm_ref[ti * t:(ti + 1) * t, :] = dm
        rmax = jnp.maximum(rmax, jnp.max(dm))
        work = dm
        ssum = jnp.zeros((t,), f32)
        for _ in range(k):
            v = jnp.min(work, axis=1)
            a = jnp.argmin(work, axis=1)
            ssum = ssum + v * v
            work = jnp.where(lane_t == a[:, None], jnp.inf, work)
        dens_parts.append(jnp.exp(-(ssum / np.float32(k))))

    dens = jnp.concatenate(dens_parts) + noise_ref[0, 0]
    dist_max = rmax

    score_parts = []
    for ti in range(nt):
        dmt = dm_ref[ti * t:(ti + 1) * t, :]
        drow = dens[ti * t:(ti + 1) * t]
        cand = jnp.where(dens[None, :] > drow[:, None], dmt, dist_max)
        score_parts.append(jnp.min(cand, axis=1) * drow)
    score = jnp.concatenate(score_parts).reshape(1, n)

    lane_n = jax.lax.broadcasted_iota(jnp.int32, (1, n), 1)
    neg = np.float32(-np.inf)

    def sel_body(j, carry):
        sw, bv, bi = carry
        i = jnp.argmax(sw)
        row = dm_ref[pl.ds(i, 1), :]
        lt = row < bv
        bv = jnp.where(lt, row, bv)
        bi = jnp.where(lt, j, bi)
        sel = lane_n == i
        bv = jnp.where(sel, neg, bv)
        bi = jnp.where(sel, j, bi)
        sw = jnp.where(sel, neg, sw)
        return sw, bv, bi

    bv0 = jnp.full((1, n), jnp.inf, f32)
    bi0 = jnp.zeros((1, n), jnp.int32)
    _, _, bi = jax.lax.fori_loop(0, cn, sel_body, (score, bv0, bi0))

    hi = jax.lax.Precision.HIGHEST
    oh = (jax.lax.broadcasted_iota(jnp.int32, (cn, n), 0) == bi).astype(f32)
    ws = jnp.exp(jax.lax.dot_general(xb, wt_ref[...], (((1,), (0,)), ((), ())),
                                     precision=hi, preferred_element_type=f32)
                 + b_ref[0, 0])
    wsum = jax.lax.dot_general(oh, ws, (((1,), (0,)), ((), ())),
                               precision=hi, preferred_element_type=f32)
    wsum = wsum + np.float32(1e-6)
    msum = jax.lax.dot_general(oh, xb * ws, (((1,), (0,)), ((), ())),
                               precision=hi, preferred_element_type=f32)
    out_ref[0] = msum / wsum


def _ctm_call(x, wt, b, noise, *, cn, k, t, prec_g, interpret=False):
    bsz, n, c = x.shape
    body = functools.partial(_ctm_body, n=n, c=c, cn=cn, k=k, t=t,
                             prec_g=prec_g)
    return pl.pallas_call(
        body,
        grid=(bsz,),
        in_specs=[
            pl.BlockSpec((1, n, c), lambda i: (i, 0, 0)),
            pl.BlockSpec((c, 1), lambda i: (0, 0)),
            pl.BlockSpec((1, 1), lambda i: (0, 0)),
            pl.BlockSpec((1, 1, n), lambda i: (i, 0, 0)),
        ],
        out_specs=pl.BlockSpec((1, cn, c), lambda i: (i, 0, 0)),
        out_shape=jax.ShapeDtypeStruct((bsz, cn, c), jnp.float32),
        scratch_shapes=[pltpu.VMEM((n, n), jnp.float32)],
        compiler_params=pltpu.CompilerParams(
            dimension_semantics=("arbitrary",)),
        interpret=interpret,
    )(x, wt, b, noise)


def kernel(x, W_score, b_score):
    bsz, n, _ = x.shape
    noise = jax.random.uniform(jax.random.key(1), (bsz, n),
                               dtype=jnp.float32) * 1e-06
    return _ctm_call(
        x, W_score.T, b_score.reshape(1, 1), noise.reshape(bsz, 1, n),
        cn=_CN, k=_K, t=_T, prec_g=jax.lax.Precision.DEFAULT)


# speedup vs baseline: 5.4276x; 1.0006x over previous
import functools

import jax
import jax.numpy as jnp
import numpy as np
from jax.experimental import pallas as pl
from jax.experimental.pallas import tpu as pltpu

_B, _N, _C = 8, 1568, 384
_CN, _K = 196, 9
_T = 224


def _ctm_body(x_ref, wt_ref, b_ref, noise_ref, out_ref, dm_ref, *,
              n, c, cn, k, t, prec_g):
    f32 = jnp.float32
    sqrt_c = np.float32(c ** 0.5)
    xb = x_ref[0]
    sq = jnp.sum(xb * xb, axis=1)

    nt = n // t
    lane_t = jax.lax.broadcasted_iota(jnp.int32, (t, n), 1)

    dens_parts = []
    rmax = jnp.full((), -jnp.inf, f32)
    for ti in range(nt):
        rows = xb[ti * t:(ti + 1) * t]
        g = jax.lax.dot_general(rows, xb, (((1,), (1,)), ((), ())),
                                precision=prec_g, preferred_element_type=f32)
        d2 = sq[ti * t:(ti + 1) * t][:, None] + sq[None, :] - 2.0 * g
        dm = jnp.sqrt(jnp.maximum(d2, 0.0)) / sqrt_c
        dm_ref[ti * t:(ti + 1) * t, :] = dm
        rmax = jnp.maximum(rmax, jnp.max(dm))
        work = dm
        ssum = jnp.zeros((t,), f32)
        for _ in range(k):
            v = jnp.min(work, axis=1)
            a = jnp.argmin(work, axis=1)
            ssum = ssum + v * v
            work = jnp.where(lane_t == a[:, None], jnp.inf, work)
        dens_parts.append(jnp.exp(-(ssum / np.float32(k))))

    dens = jnp.concatenate(dens_parts) + noise_ref[0, 0]
    dist_max = rmax

    score_parts = []
    for ti in range(nt):
        dmt = dm_ref[ti * t:(ti + 1) * t, :]
        drow = dens[ti * t:(ti + 1) * t]
        cand = jnp.where(dens[None, :] > drow[:, None], dmt, dist_max)
        score_parts.append(jnp.min(cand, axis=1) * drow)
    score = jnp.concatenate(score_parts).reshape(1, n)

    lane_n = jax.lax.broadcasted_iota(jnp.int32, (1, n), 1)
    neg = np.float32(-np.inf)

    def sel_body(j, carry):
        sw, bv, bi = carry
        i = jnp.argmax(sw)
        row = dm_ref[pl.ds(i, 1), :]
        lt = row < bv
        bv = jnp.where(lt, row, bv)
        bi = jnp.where(lt, j, bi)
        sel = lane_n == i
        bv = jnp.where(sel, neg, bv)
        bi = jnp.where(sel, j, bi)
        sw = jnp.where(sel, neg, sw)
        return sw, bv, bi

    bv0 = jnp.full((1, n), jnp.inf, f32)
    bi0 = jnp.zeros((1, n), jnp.int32)
    _, _, bi = jax.lax.fori_loop(0, cn, sel_body, (score, bv0, bi0))

    hi = jax.lax.Precision.HIGHEST
    oh = (jax.lax.broadcasted_iota(jnp.int32, (cn, n), 0) == bi).astype(f32)
    ws = jnp.exp(jax.lax.dot_general(xb, wt_ref[...], (((1,), (0,)), ((), ())),
                                     precision=hi, preferred_element_type=f32)
                 + b_ref[0, 0])
    wsum = jax.lax.dot_general(oh, ws, (((1,), (0,)), ((), ())),
                               precision=hi, preferred_element_type=f32)
    wsum = wsum + np.float32(1e-6)
    msum = jax.lax.dot_general(oh, xb * ws, (((1,), (0,)), ((), ())),
                               precision=hi, preferred_element_type=f32)
    out_ref[0] = msum / wsum


def _ctm_call(x, wt, b, noise, *, cn, k, t, prec_g, interpret=False):
    bsz, n, c = x.shape
    body = functools.partial(_ctm_body, n=n, c=c, cn=cn, k=k, t=t,
                             prec_g=prec_g)
    return pl.pallas_call(
        body,
        grid=(bsz,),
        in_specs=[
            pl.BlockSpec((1, n, c), lambda i: (i, 0, 0)),
            pl.BlockSpec((c, 1), lambda i: (0, 0)),
            pl.BlockSpec((1, 1), lambda i: (0, 0)),
            pl.BlockSpec((1, 1, n), lambda i: (i, 0, 0)),
        ],
        out_specs=pl.BlockSpec((1, cn, c), lambda i: (i, 0, 0)),
        out_shape=jax.ShapeDtypeStruct((bsz, cn, c), jnp.float32),
        scratch_shapes=[pltpu.VMEM((n, n), jnp.float32)],
        compiler_params=pltpu.CompilerParams(
            dimension_semantics=("parallel",)),
        interpret=interpret,
    )(x, wt, b, noise)


def kernel(x, W_score, b_score):
    bsz, n, _ = x.shape
    noise = jax.random.uniform(jax.random.key(1), (bsz, n),
                               dtype=jnp.float32) * 1e-06
    return _ctm_call(
        x, W_score.T, b_score.reshape(1, 1), noise.reshape(bsz, 1, n),
        cn=_CN, k=_K, t=_T, prec_g=jax.lax.Precision.DEFAULT)


# speedup vs baseline: 6.9837x; 1.2867x over previous
import functools

import jax
import jax.numpy as jnp
import numpy as np
from jax.experimental import pallas as pl
from jax.experimental.pallas import tpu as pltpu

_B, _N, _C = 8, 1568, 384
_CN, _K = 196, 9
_T = 224


def _ctm_body(x_ref, wt_ref, b_ref, noise_ref, out_ref, dm_ref, *,
              n, c, cn, k, t, prec_g):
    f32 = jnp.float32
    sqrt_c = np.float32(c ** 0.5)
    xb = x_ref[0]
    sq = jnp.sum(xb * xb, axis=1)

    nt = n // t

    dens_parts = []
    rmax = jnp.full((), -jnp.inf, f32)
    for ti in range(nt):
        rows = xb[ti * t:(ti + 1) * t]
        g = jax.lax.dot_general(rows, xb, (((1,), (1,)), ((), ())),
                                precision=prec_g, preferred_element_type=f32)
        d2 = sq[ti * t:(ti + 1) * t][:, None] + sq[None, :] - 2.0 * g
        dm = jnp.sqrt(jnp.maximum(d2, 0.0)) / sqrt_c
        dm_ref[ti * t:(ti + 1) * t, :] = dm
        rmax = jnp.maximum(rmax, jnp.max(dm))
        work = dm
        ssum = jnp.zeros((t,), f32)
        rem = jnp.full((t,), k, jnp.int32)
        for it in range(k):
            v = jnp.min(work, axis=1)
            v2 = v * v
            if it < k - 1:
                eq = work == v[:, None]
                cnt = jnp.sum(eq.astype(jnp.int32), axis=1)
                work = jnp.where(eq, jnp.inf, work)
                take = jnp.minimum(cnt, rem)
            else:
                take = rem
            for j in range(k - it):
                ssum = ssum + jnp.where(j < take, v2, np.float32(0.0))
            rem = rem - take
        dens_parts.append(jnp.exp(-(ssum / np.float32(k))))

    dens = jnp.concatenate(dens_parts) + noise_ref[0, 0]
    dist_max = rmax

    score_parts = []
    for ti in range(nt):
        dmt = dm_ref[ti * t:(ti + 1) * t, :]
        drow = dens[ti * t:(ti + 1) * t]
        cand = jnp.where(dens[None, :] > drow[:, None], dmt, dist_max)
        score_parts.append(jnp.min(cand, axis=1) * drow)
    score = jnp.concatenate(score_parts).reshape(1, n)

    lane_n = jax.lax.broadcasted_iota(jnp.int32, (1, n), 1)
    neg = np.float32(-np.inf)

    def sel_body(j, carry):
        sw, bv, bi = carry
        i = jnp.argmax(sw)
        row = dm_ref[pl.ds(i, 1), :]
        lt = row < bv
        bv = jnp.where(lt, row, bv)
        bi = jnp.where(lt, j, bi)
        sel = lane_n == i
        bv = jnp.where(sel, neg, bv)
        bi = jnp.where(sel, j, bi)
        sw = jnp.where(sel, neg, sw)
        return sw, bv, bi

    bv0 = jnp.full((1, n), jnp.inf, f32)
    bi0 = jnp.zeros((1, n), jnp.int32)
    _, _, bi = jax.lax.fori_loop(0, cn, sel_body, (score, bv0, bi0))

    hi = jax.lax.Precision.HIGHEST
    oh = (jax.lax.broadcasted_iota(jnp.int32, (cn, n), 0) == bi).astype(f32)
    ws = jnp.exp(jax.lax.dot_general(xb, wt_ref[...], (((1,), (0,)), ((), ())),
                                     precision=hi, preferred_element_type=f32)
                 + b_ref[0, 0])
    wsum = jax.lax.dot_general(oh, ws, (((1,), (0,)), ((), ())),
                               precision=hi, preferred_element_type=f32)
    wsum = wsum + np.float32(1e-6)
    msum = jax.lax.dot_general(oh, xb * ws, (((1,), (0,)), ((), ())),
                               precision=hi, preferred_element_type=f32)
    out_ref[0] = msum / wsum


def _ctm_call(x, wt, b, noise, *, cn, k, t, prec_g, interpret=False):
    bsz, n, c = x.shape
    body = functools.partial(_ctm_body, n=n, c=c, cn=cn, k=k, t=t,
                             prec_g=prec_g)
    return pl.pallas_call(
        body,
        grid=(bsz,),
        in_specs=[
            pl.BlockSpec((1, n, c), lambda i: (i, 0, 0)),
            pl.BlockSpec((c, 1), lambda i: (0, 0)),
            pl.BlockSpec((1, 1), lambda i: (0, 0)),
            pl.BlockSpec((1, 1, n), lambda i: (i, 0, 0)),
        ],
        out_specs=pl.BlockSpec((1, cn, c), lambda i: (i, 0, 0)),
        out_shape=jax.ShapeDtypeStruct((bsz, cn, c), jnp.float32),
        scratch_shapes=[pltpu.VMEM((n, n), jnp.float32)],
        compiler_params=pltpu.CompilerParams(
            dimension_semantics=("parallel",)),
        interpret=interpret,
    )(x, wt, b, noise)


def kernel(x, W_score, b_score):
    bsz, n, _ = x.shape
    noise = jax.random.uniform(jax.random.key(1), (bsz, n),
                               dtype=jnp.float32) * 1e-06
    return _ctm_call(
        x, W_score.T, b_score.reshape(1, 1), noise.reshape(bsz, 1, n),
        cn=_CN, k=_K, t=_T, prec_g=jax.lax.Precision.DEFAULT)


# speedup vs baseline: 9.3988x; 1.3458x over previous
import functools

import jax
import jax.numpy as jnp
import numpy as np
from jax.experimental import pallas as pl
from jax.experimental.pallas import tpu as pltpu

_B, _N, _C = 8, 1568, 384
_CN, _K = 196, 9
_T = 224
_BLK = 2


def _ctm_body(x_ref, wt_ref, b_ref, noise_ref, out_ref, dm_ref, *,
              n, c, cn, k, t, blk, prec_g):
    f32 = jnp.float32
    sqrt_c = np.float32(c ** 0.5)
    nt = n // t

    score_rows = []
    for g in range(blk):
        xb = x_ref[g]
        sq = jnp.sum(xb * xb, axis=1)

        dens_parts = []
        rmax = jnp.full((), -jnp.inf, f32)
        for ti in range(nt):
            rows = xb[ti * t:(ti + 1) * t]
            g_mm = jax.lax.dot_general(rows, xb, (((1,), (1,)), ((), ())),
                                       precision=prec_g,
                                       preferred_element_type=f32)
            d2 = sq[ti * t:(ti + 1) * t][:, None] + sq[None, :] - 2.0 * g_mm
            dm = jnp.sqrt(jnp.maximum(d2, 0.0)) / sqrt_c
            dm_ref[g, ti * t:(ti + 1) * t, :] = dm
            rmax = jnp.maximum(rmax, jnp.max(dm))
            work = dm
            ssum = jnp.zeros((t,), f32)
            rem = jnp.full((t,), k, jnp.int32)
            for it in range(k):
                v = jnp.min(work, axis=1)
                v2 = v * v
                if it < k - 1:
                    eq = work == v[:, None]
                    cnt = jnp.sum(eq.astype(jnp.int32), axis=1)
                    work = jnp.where(eq, jnp.inf, work)
                    take = jnp.minimum(cnt, rem)
                else:
                    take = rem
                for j in range(k - it):
                    ssum = ssum + jnp.where(j < take, v2, np.float32(0.0))
                rem = rem - take
            dens_parts.append(jnp.exp(-(ssum / np.float32(k))))

        dens = jnp.concatenate(dens_parts) + noise_ref[g, 0]
        dist_max = rmax

        score_parts = []
        for ti in range(nt):
            dmt = dm_ref[g, ti * t:(ti + 1) * t, :]
            drow = dens[ti * t:(ti + 1) * t]
            cand = jnp.where(dens[None, :] > drow[:, None], dmt, dist_max)
            score_parts.append(jnp.min(cand, axis=1) * drow)
        score_rows.append(jnp.concatenate(score_parts).reshape(1, n))

    score = jnp.concatenate(score_rows, axis=0)

    lane_n = jax.lax.broadcasted_iota(jnp.int32, (blk, n), 1)
    neg = np.float32(-np.inf)

    def sel_body(j, carry):
        sw, bv, bi = carry
        i = jnp.argmax(sw, axis=1)
        row = jnp.concatenate(
            [dm_ref[g, pl.ds(i[g], 1), :] for g in range(blk)], axis=0)
        lt = row < bv
        bv = jnp.where(lt, row, bv)
        bi = jnp.where(lt, j, bi)
        sel = lane_n == i[:, None]
        bv = jnp.where(sel, neg, bv)
        bi = jnp.where(sel, j, bi)
        sw = jnp.where(sel, neg, sw)
        return sw, bv, bi

    bv0 = jnp.full((blk, n), jnp.inf, f32)
    bi0 = jnp.zeros((blk, n), jnp.int32)
    _, _, bi = jax.lax.fori_loop(0, cn, sel_body, (score, bv0, bi0))

    hi = jax.lax.Precision.HIGHEST
    for g in range(blk):
        xb = x_ref[g]
        oh = (jax.lax.broadcasted_iota(jnp.int32, (cn, n), 0)
              == bi[g][None, :]).astype(f32)
        ws = jnp.exp(
            jax.lax.dot_general(xb, wt_ref[...], (((1,), (0,)), ((), ())),
                                precision=hi, preferred_element_type=f32)
            + b_ref[0, 0])
        wsum = jax.lax.dot_general(oh, ws, (((1,), (0,)), ((), ())),
                                   precision=hi, preferred_element_type=f32)
        wsum = wsum + np.float32(1e-6)
        msum = jax.lax.dot_general(oh, xb * ws, (((1,), (0,)), ((), ())),
                                   precision=hi, preferred_element_type=f32)
        out_ref[g] = msum / wsum


def _ctm_call(x, wt, b, noise, *, cn, k, t, blk, prec_g, interpret=False):
    bsz, n, c = x.shape
    body = functools.partial(_ctm_body, n=n, c=c, cn=cn, k=k, t=t, blk=blk,
                             prec_g=prec_g)
    return pl.pallas_call(
        body,
        grid=(bsz // blk,),
        in_specs=[
            pl.BlockSpec((blk, n, c), lambda i: (i, 0, 0)),
            pl.BlockSpec((c, 1), lambda i: (0, 0)),
            pl.BlockSpec((1, 1), lambda i: (0, 0)),
            pl.BlockSpec((blk, 1, n), lambda i: (i, 0, 0)),
        ],
        out_specs=pl.BlockSpec((blk, cn, c), lambda i: (i, 0, 0)),
        out_shape=jax.ShapeDtypeStruct((bsz, cn, c), jnp.float32),
        scratch_shapes=[pltpu.VMEM((blk, n, n), jnp.float32)],
        compiler_params=pltpu.CompilerParams(
            dimension_semantics=("arbitrary",)),
        interpret=interpret,
    )(x, wt, b, noise)


def kernel(x, W_score, b_score):
    bsz, n, _ = x.shape
    noise = jax.random.uniform(jax.random.key(1), (bsz, n),
                               dtype=jnp.float32) * 1e-06
    return _ctm_call(
        x, W_score.T, b_score.reshape(1, 1), noise.reshape(bsz, 1, n),
        cn=_CN, k=_K, t=_T, blk=_BLK, prec_g=jax.lax.Precision.DEFAULT)
